# baseline (speedup 1.0000x reference)
import functools

import jax
import jax.numpy as jnp
from jax import lax
from jax.experimental import pallas as pl
from jax.experimental.pallas import tpu as pltpu
from jax.experimental.pallas import tpu_sc as plsc

N = 50000
NPAD = 50048
NG = 64
E = 800000
CHUNK = 256
NW = 32
NBUF = 3
NCHT0 = 66
NCHT1 = 132
NCHM = max(NCHT0, NCHT1)
E0 = 16 * NCHT0 * CHUNK
E1 = 16 * NCHT1 * CHUNK
NTILES = 16
RPT = NPAD // NTILES
_SQRT_HALF = 0.7071067811865476


def _gelu(t):
    return 0.5 * t * (1.0 + lax.erf(t * _SQRT_HALF))


def _sc_mesh():
    return plsc.VectorSubcoreMesh(core_axis_name="c", subcore_axis_name="s",
                                  num_cores=2, num_subcores=NTILES)


@functools.cache
def _make_sc_degree():
    return functools.partial(
        pl.kernel,
        out_type=jax.ShapeDtypeStruct((2, NPAD), jnp.float32),
        mesh=_sc_mesh(),
        scratch_types=[
            pltpu.VMEM((NCHM, CHUNK), jnp.int32),
            pltpu.VMEM((CHUNK,), jnp.float32),
            pltpu.VMEM_SHARED((NPAD,), jnp.float32),
        ] + [pltpu.SemaphoreType.DMA] * NBUF,
        compiler_params=pltpu.CompilerParams(use_tc_tiling_on_sc=False),
    )(_sc_degree_body)


def _sc_degree_body(dst_hbm, zeros_hbm, out_hbm, dst_v, ones_v, acc, *ssems):
    c = lax.axis_index("c")
    s = lax.axis_index("s")
    wid = c * NTILES + s
    base = s * RPT
    nblk = jnp.where(c == 0, NCHT0 // NBUF, NCHT1 // NBUF)
    pltpu.sync_copy(zeros_hbm.at[pl.ds(base, RPT)], acc.at[pl.ds(base, RPT)])
    pltpu.sync_copy(dst_hbm.at[wid], dst_v)
    for i in range(CHUNK // 16):
        ones_v[pl.ds(i * 16, 16)] = jnp.ones((16,), jnp.float32)
    plsc.subcore_barrier()

    def body(i, carry):
        j0 = i * NBUF
        for b in range(NBUF):
            pltpu.async_copy(ones_v, acc.at[dst_v.at[j0 + b]], ssems[b],
                             add=True)
        for b in range(NBUF):
            pltpu.make_async_copy(ones_v, acc.at[dst_v.at[j0 + b]],
                                  ssems[b]).wait()
        return carry

    lax.fori_loop(0, nblk, body, 0)
    plsc.subcore_barrier()
    pltpu.sync_copy(acc.at[pl.ds(base, RPT)], out_hbm.at[c, pl.ds(base, RPT)])


FP = 16


@functools.cache
def _make_sc_prop():
    feat = (FP,)

    @functools.partial(
        pl.kernel,
        out_type=jax.ShapeDtypeStruct((2, NPAD) + feat, jnp.float32),
        mesh=_sc_mesh(),
        scratch_types=[
            pltpu.VMEM((NCHM, CHUNK), jnp.int32),
            pltpu.VMEM((NCHM, CHUNK), jnp.int32),
            pltpu.VMEM_SHARED((NPAD,) + feat, jnp.float32),
        ] + [pltpu.VMEM((CHUNK,) + feat, jnp.float32)] * NBUF
          + [pltpu.SemaphoreType.DMA] * (2 * NBUF),
        compiler_params=pltpu.CompilerParams(use_tc_tiling_on_sc=False),
    )
    def prop(src_hbm, dst_hbm, g_hbm, zeros_hbm, out_hbm,
             src_v, dst_v, acc, *rest):
        bufs = rest[:NBUF]
        gsems = rest[NBUF:2 * NBUF]
        ssems = rest[2 * NBUF:3 * NBUF]
        c = lax.axis_index("c")
        s = lax.axis_index("s")
        wid = c * NTILES + s
        base = s * RPT
        nblk = jnp.where(c == 0, NCHT0 // NBUF, NCHT1 // NBUF)
        pltpu.sync_copy(zeros_hbm.at[pl.ds(base, RPT)],
                        acc.at[pl.ds(base, RPT)])
        pltpu.sync_copy(src_hbm.at[wid], src_v)
        pltpu.sync_copy(dst_hbm.at[wid], dst_v)
        plsc.subcore_barrier()

        for b in range(NBUF):
            pltpu.async_copy(g_hbm.at[src_v.at[b]], bufs[b], gsems[b])

        def body(i, carry):
            j0 = i * NBUF
            for b in range(NBUF):
                pltpu.make_async_copy(g_hbm.at[src_v.at[j0 + b]],
                                      bufs[b], gsems[b]).wait()
                pltpu.async_copy(bufs[b], acc.at[dst_v.at[j0 + b]],
                                 ssems[b], add=True)
            for b in range(NBUF):
                pltpu.make_async_copy(bufs[b], acc.at[dst_v.at[j0 + b]],
                                      ssems[b]).wait()
                pltpu.async_copy(g_hbm.at[src_v.at[j0 + NBUF + b]],
                                 bufs[b], gsems[b])
            return carry

        lax.fori_loop(0, nblk - 1, body, 0)
        j0 = (nblk - 1) * NBUF
        for b in range(NBUF):
            pltpu.make_async_copy(g_hbm.at[src_v.at[j0 + b]],
                                  bufs[b], gsems[b]).wait()
            pltpu.async_copy(bufs[b], acc.at[dst_v.at[j0 + b]],
                             ssems[b], add=True)
        for b in range(NBUF):
            pltpu.make_async_copy(bufs[b], acc.at[dst_v.at[j0 + b]],
                                  ssems[b]).wait()
        plsc.subcore_barrier()
        pltpu.sync_copy(acc.at[pl.ds(base, RPT)],
                        out_hbm.at[c, pl.ds(base, RPT)])

    return prop


def _lane_mask():
    i = lax.broadcasted_iota(jnp.int32, (1, NPAD), 1)
    return (i < N).astype(jnp.float32)


def _k0_body(deg_ref, x_ref, dinv_ref, g1_ref):
    deg = deg_ref[0] + deg_ref[1] + 1.0
    dinv = lax.rsqrt(deg) * _lane_mask()
    dinv_ref[...] = dinv
    g1_ref[...] = jnp.concatenate(
        [dinv * x_ref[...], jnp.zeros((FP - 1, NPAD), jnp.float32)], axis=0)


_k0 = pl.pallas_call(
    _k0_body,
    out_shape=(jax.ShapeDtypeStruct((1, NPAD), jnp.float32),
               jax.ShapeDtypeStruct((FP, NPAD), jnp.float32)),
)


def _bn_scale(h, use_bn, gm_ref, be_ref, dinv):
    if not use_bn:
        return h
    msk = _lane_mask()
    mu = jnp.sum(h * msk, axis=1, keepdims=True) * (1.0 / N)
    d = (h - mu) * msk
    var = jnp.sum(d * d, axis=1, keepdims=True) * (1.0 / N)
    h = (h - mu) * lax.rsqrt(var + 1e-5) * gm_ref[...] + be_ref[...]
    return h * dinv


def _layer_body(fi, fo, use_bn,
                p_ref, g_ref, dinv_ref, wt_ref, b_ref, gm_ref, be_ref,
                out_ref):
    agg = (p_ref[0, :fi] + p_ref[1, :fi] + g_ref[:fi]) * dinv_ref[...]
    if fi == 1:
        c = wt_ref[...] * agg
    else:
        c = lax.dot_general(wt_ref[...], agg, (((1,), (0,)), ((), ())),
                            preferred_element_type=jnp.float32)
    h = _gelu(c + b_ref[...])
    h = _bn_scale(h, use_bn, gm_ref, be_ref, dinv_ref[...])
    if fo < FP:
        h = jnp.concatenate(
            [h, jnp.zeros((FP - fo, NPAD), jnp.float32)], axis=0)
    out_ref[...] = h


def _make_tc_layer(fi, fo, use_bn):
    return pl.pallas_call(
        functools.partial(_layer_body, fi, fo, use_bn),
        out_shape=jax.ShapeDtypeStruct((max(fo, FP), NPAD), jnp.float32),
    )


def _layer4_body(pa_ref, pb_ref, g_ref, dinv_ref, wat_ref, wbt_ref, b_ref,
                 out_ref):
    dinv = dinv_ref[...]
    agg_a = (pa_ref[0] + pa_ref[1] + g_ref[:FP]) * dinv
    agg_b = (pb_ref[0] + pb_ref[1] + g_ref[FP:]) * dinv
    mm = lambda w, a: lax.dot_general(w, a, (((1,), (0,)), ((), ())),
                                      preferred_element_type=jnp.float32)
    out_ref[...] = _gelu(mm(wat_ref[...], agg_a) + mm(wbt_ref[...], agg_b)
                         + b_ref[...])


_tc_layer4 = pl.pallas_call(
    _layer4_body,
    out_shape=jax.ShapeDtypeStruct((64, NPAD), jnp.float32),
)

_TC_LAYER = [_make_tc_layer(1, 8, True), _make_tc_layer(8, 16, True),
             _make_tc_layer(16, 32, True)]


def _pool_body(h4_ref, batch_ref, y_ref,
               l1a_ref, l1b_ref, l1bias_ref, l2w_ref, l2b_ref,
               l3w_ref, l3b_ref, l4w_ref, l4b_ref, out_ref):
    gi = lax.broadcasted_iota(jnp.int32, (NG, 1), 0)
    onehot = (batch_ref[...] == gi).astype(jnp.float32)
    pooled = lax.dot_general(onehot, h4_ref[...], (((1,), (1,)), ((), ())),
                             preferred_element_type=jnp.float32)
    cnt = jnp.sum(onehot, axis=1, keepdims=True)
    z = pooled / jnp.maximum(cnt, 1.0)
    mm = lambda a, w: lax.dot_general(a, w, (((1,), (0,)), ((), ())),
                                      preferred_element_type=jnp.float32)
    z = _gelu(mm(z, l1a_ref[...]) + mm(y_ref[...], l1b_ref[...])
              + l1bias_ref[...])
    z = _gelu(mm(z, l2w_ref[...]) + l2b_ref[...])
    z = _gelu(mm(z, l3w_ref[...]) + l3b_ref[...])
    out_ref[...] = jax.nn.sigmoid(mm(z, l4w_ref[...]) + l4b_ref[...])


_pool = pl.pallas_call(
    _pool_body,
    out_shape=jax.ShapeDtypeStruct((NG, 2), jnp.float32),
)


def kernel(x, edge_index, batch, y_extra, W1, b1, W2, b2, W3, b3, W4, b4,
           g1, be1, g2, be2, g3, be3, L1w, L1b, L2w, L2b, L3w, L3b, L4w, L4b):
    def split_edges(v):
        a = v[:E0].reshape(NTILES, NCHT0, CHUNK)
        a = jnp.pad(a, ((0, 0), (0, NCHM - NCHT0), (0, 0)),
                    constant_values=N)
        b = jnp.concatenate([v[E0:], jnp.full((E1 - (E - E0),), N,
                                              jnp.int32)])
        b = b.reshape(NTILES, NCHT1, CHUNK)
        b = jnp.pad(b, ((0, 0), (0, NCHM - NCHT1), (0, 0)),
                    constant_values=N)
        return jnp.concatenate([a, b], axis=0)

    srcp = split_edges(edge_index[0])
    dstp = split_edges(edge_index[1])
    xp = jnp.pad(x[:, 0], (0, NPAD - N)).reshape(1, NPAD)
    batchp = jnp.pad(batch, (0, NPAD - N),
                     constant_values=NG).reshape(1, NPAD)
    zeros1 = jnp.zeros((NPAD,), jnp.float32)
    zeros16 = jnp.zeros((NPAD, FP), jnp.float32)
    prop = _make_sc_prop()

    deg2 = _make_sc_degree()(dstp, zeros1)
    dinv, gcur = _k0(deg2.reshape(2, 1, NPAD), xp)

    specs = [(1, 8, W1, b1, g1, be1), (8, 16, W2, b2, g2, be2),
             (16, 32, W3, b3, g3, be3)]
    for i, (fi, fo, W, b, gm, be) in enumerate(specs):
        p = prop(srcp, dstp, gcur.T, zeros16)
        gcur = _TC_LAYER[i](jnp.transpose(p, (0, 2, 1)), gcur, dinv, W.T,
                            b.reshape(fo, 1), gm.reshape(fo, 1),
                            be.reshape(fo, 1))

    pa = prop(srcp, dstp, gcur[:FP].T, zeros16)
    pb = prop(srcp, dstp, gcur[FP:].T, zeros16)
    W4t = W4.T
    h4 = _tc_layer4(jnp.transpose(pa, (0, 2, 1)),
                    jnp.transpose(pb, (0, 2, 1)),
                    gcur, dinv, W4t[:, :FP], W4t[:, FP:],
                    b4.reshape(64, 1))

    return _pool(h4, batchp, y_extra,
                 L1w[:NG], L1w[NG:], L1b.reshape(1, -1),
                 L2w, L2b.reshape(1, -1),
                 L3w, L3b.reshape(1, -1),
                 L4w, L4b.reshape(1, -1))


# speedup vs baseline: 1.4092x; 1.4092x over previous
import functools

import jax
import jax.numpy as jnp
from jax import lax
from jax.experimental import pallas as pl
from jax.experimental.pallas import tpu as pltpu
from jax.experimental.pallas import tpu_sc as plsc

N = 50000
NPAD = 50048
NG = 64
E = 800000
CHUNK = 128
NW = 32
NBUF = 4
NCHT0 = 108
NCHT1 = 284
NCHM = max(NCHT0, NCHT1)
E0 = 16 * NCHT0 * CHUNK
E1 = 16 * NCHT1 * CHUNK
NTILES = 16
RPT = NPAD // NTILES
_SQRT_HALF = 0.7071067811865476


def _gelu(t):
    return 0.5 * t * (1.0 + lax.erf(t * _SQRT_HALF))


def _sc_mesh():
    return plsc.VectorSubcoreMesh(core_axis_name="c", subcore_axis_name="s",
                                  num_cores=2, num_subcores=NTILES)


@functools.cache
def _make_sc_degree():
    return functools.partial(
        pl.kernel,
        out_type=jax.ShapeDtypeStruct((2, NPAD), jnp.float32),
        mesh=_sc_mesh(),
        scratch_types=[
            pltpu.VMEM((NCHM, CHUNK), jnp.int32),
            pltpu.VMEM((CHUNK,), jnp.float32),
            pltpu.VMEM_SHARED((NPAD,), jnp.float32),
        ] + [pltpu.SemaphoreType.DMA] * NBUF,
        compiler_params=pltpu.CompilerParams(use_tc_tiling_on_sc=False),
    )(_sc_degree_body)


def _sc_degree_body(dst_hbm, zeros_hbm, out_hbm, dst_v, ones_v, acc, *ssems):
    c = lax.axis_index("c")
    s = lax.axis_index("s")
    wid = c * NTILES + s
    base = s * RPT
    nblk = jnp.where(c == 0, NCHT0 // NBUF, NCHT1 // NBUF)
    pltpu.sync_copy(zeros_hbm.at[pl.ds(base, RPT)], acc.at[pl.ds(base, RPT)])
    pltpu.sync_copy(dst_hbm.at[wid], dst_v)
    for i in range(CHUNK // 16):
        ones_v[pl.ds(i * 16, 16)] = jnp.ones((16,), jnp.float32)
    plsc.subcore_barrier()

    def body(i, carry):
        j0 = i * NBUF
        for b in range(NBUF):
            pltpu.async_copy(ones_v, acc.at[dst_v.at[j0 + b]], ssems[b],
                             add=True)
        for b in range(NBUF):
            pltpu.make_async_copy(ones_v, acc.at[dst_v.at[j0 + b]],
                                  ssems[b]).wait()
        return carry

    lax.fori_loop(0, nblk, body, 0)
    plsc.subcore_barrier()
    pltpu.sync_copy(acc.at[pl.ds(base, RPT)], out_hbm.at[c, pl.ds(base, RPT)])


FP = 16


@functools.cache
def _make_sc_prop(F=FP, dtype=jnp.float32):
    feat = (F,)

    @functools.partial(
        pl.kernel,
        out_type=jax.ShapeDtypeStruct((2, NPAD) + feat, dtype),
        mesh=_sc_mesh(),
        scratch_types=[
            pltpu.VMEM((NCHM, CHUNK), jnp.int32),
            pltpu.VMEM((NCHM, CHUNK), jnp.int32),
            pltpu.VMEM_SHARED((NPAD,) + feat, dtype),
        ] + [pltpu.VMEM((CHUNK,) + feat, dtype)] * NBUF
          + [pltpu.SemaphoreType.DMA] * (2 * NBUF),
        compiler_params=pltpu.CompilerParams(use_tc_tiling_on_sc=False),
    )
    def prop(src_hbm, dst_hbm, g_hbm, zeros_hbm, out_hbm,
             src_v, dst_v, acc, *rest):
        bufs = rest[:NBUF]
        gsems = rest[NBUF:2 * NBUF]
        ssems = rest[2 * NBUF:3 * NBUF]
        c = lax.axis_index("c")
        s = lax.axis_index("s")
        wid = c * NTILES + s
        base = s * RPT
        nblk = jnp.where(c == 0, NCHT0 // NBUF, NCHT1 // NBUF)
        pltpu.sync_copy(zeros_hbm.at[pl.ds(base, RPT)],
                        acc.at[pl.ds(base, RPT)])
        pltpu.sync_copy(src_hbm.at[wid], src_v)
        pltpu.sync_copy(dst_hbm.at[wid], dst_v)
        plsc.subcore_barrier()

        for b in range(NBUF):
            pltpu.async_copy(g_hbm.at[src_v.at[b]], bufs[b], gsems[b])

        def body(i, carry):
            j0 = i * NBUF
            for b in range(NBUF):
                pltpu.make_async_copy(g_hbm.at[src_v.at[j0 + b]],
                                      bufs[b], gsems[b]).wait()
                pltpu.async_copy(bufs[b], acc.at[dst_v.at[j0 + b]],
                                 ssems[b], add=True)
            for b in range(NBUF):
                pltpu.make_async_copy(bufs[b], acc.at[dst_v.at[j0 + b]],
                                      ssems[b]).wait()
                pltpu.async_copy(g_hbm.at[src_v.at[j0 + NBUF + b]],
                                 bufs[b], gsems[b])
            return carry

        lax.fori_loop(0, nblk - 1, body, 0)
        j0 = (nblk - 1) * NBUF
        for b in range(NBUF):
            pltpu.make_async_copy(g_hbm.at[src_v.at[j0 + b]],
                                  bufs[b], gsems[b]).wait()
            pltpu.async_copy(bufs[b], acc.at[dst_v.at[j0 + b]],
                             ssems[b], add=True)
        for b in range(NBUF):
            pltpu.make_async_copy(bufs[b], acc.at[dst_v.at[j0 + b]],
                                  ssems[b]).wait()
        plsc.subcore_barrier()
        pltpu.sync_copy(acc.at[pl.ds(base, RPT)],
                        out_hbm.at[c, pl.ds(base, RPT)])

    return prop


def _lane_mask():
    i = lax.broadcasted_iota(jnp.int32, (1, NPAD), 1)
    return (i < N).astype(jnp.float32)


def _k0_body(deg_ref, x_ref, dinv_ref, g1_ref):
    deg = deg_ref[0] + deg_ref[1] + 1.0
    dinv = lax.rsqrt(deg) * _lane_mask()
    dinv_ref[...] = dinv
    g1_ref[...] = jnp.concatenate(
        [dinv * x_ref[...], jnp.zeros((FP - 1, NPAD), jnp.float32)], axis=0)


_k0 = pl.pallas_call(
    _k0_body,
    out_shape=(jax.ShapeDtypeStruct((1, NPAD), jnp.float32),
               jax.ShapeDtypeStruct((FP, NPAD), jnp.float32)),
)


def _bn_scale(h, use_bn, gm_ref, be_ref, dinv):
    if not use_bn:
        return h
    msk = _lane_mask()
    mu = jnp.sum(h * msk, axis=1, keepdims=True) * (1.0 / N)
    d = (h - mu) * msk
    var = jnp.sum(d * d, axis=1, keepdims=True) * (1.0 / N)
    h = (h - mu) * lax.rsqrt(var + 1e-5) * gm_ref[...] + be_ref[...]
    return h * dinv


def _layer_body(fi, fo, use_bn,
                p_ref, g_ref, dinv_ref, wt_ref, b_ref, gm_ref, be_ref,
                out_ref):
    agg = (p_ref[0, :fi] + p_ref[1, :fi] + g_ref[:fi]) * dinv_ref[...]
    if fi == 1:
        c = wt_ref[...] * agg
    else:
        c = lax.dot_general(wt_ref[...], agg, (((1,), (0,)), ((), ())),
                            preferred_element_type=jnp.float32)
    h = _gelu(c + b_ref[...])
    h = _bn_scale(h, use_bn, gm_ref, be_ref, dinv_ref[...])
    if fo < FP:
        h = jnp.concatenate(
            [h, jnp.zeros((FP - fo, NPAD), jnp.float32)], axis=0)
    out_ref[...] = h


def _make_tc_layer(fi, fo, use_bn):
    return pl.pallas_call(
        functools.partial(_layer_body, fi, fo, use_bn),
        out_shape=jax.ShapeDtypeStruct((max(fo, FP), NPAD), jnp.float32),
    )


def _layer4_body(p_ref, g_ref, dinv_ref, wt_ref, b_ref, out_ref):
    pf = p_ref[...].astype(jnp.float32)
    agg = (pf[0] + pf[1] + g_ref[...]) * dinv_ref[...]
    c = lax.dot_general(wt_ref[...], agg, (((1,), (0,)), ((), ())),
                        preferred_element_type=jnp.float32)
    out_ref[...] = _gelu(c + b_ref[...])


_tc_layer4 = pl.pallas_call(
    _layer4_body,
    out_shape=jax.ShapeDtypeStruct((64, NPAD), jnp.float32),
)

_TC_LAYER = [_make_tc_layer(1, 8, True), _make_tc_layer(8, 16, True),
             _make_tc_layer(16, 32, True)]


def _pool_body(h4_ref, batch_ref, y_ref,
               l1a_ref, l1b_ref, l1bias_ref, l2w_ref, l2b_ref,
               l3w_ref, l3b_ref, l4w_ref, l4b_ref, out_ref):
    gi = lax.broadcasted_iota(jnp.int32, (NG, 1), 0)
    onehot = (batch_ref[...] == gi).astype(jnp.float32)
    pooled = lax.dot_general(onehot, h4_ref[...], (((1,), (1,)), ((), ())),
                             preferred_element_type=jnp.float32)
    cnt = jnp.sum(onehot, axis=1, keepdims=True)
    z = pooled / jnp.maximum(cnt, 1.0)
    mm = lambda a, w: lax.dot_general(a, w, (((1,), (0,)), ((), ())),
                                      preferred_element_type=jnp.float32)
    z = _gelu(mm(z, l1a_ref[...]) + mm(y_ref[...], l1b_ref[...])
              + l1bias_ref[...])
    z = _gelu(mm(z, l2w_ref[...]) + l2b_ref[...])
    z = _gelu(mm(z, l3w_ref[...]) + l3b_ref[...])
    out_ref[...] = jax.nn.sigmoid(mm(z, l4w_ref[...]) + l4b_ref[...])


_pool = pl.pallas_call(
    _pool_body,
    out_shape=jax.ShapeDtypeStruct((NG, 2), jnp.float32),
)


def kernel(x, edge_index, batch, y_extra, W1, b1, W2, b2, W3, b3, W4, b4,
           g1, be1, g2, be2, g3, be3, L1w, L1b, L2w, L2b, L3w, L3b, L4w, L4b):
    def split_edges(v):
        a = v[:E0].reshape(NTILES, NCHT0, CHUNK)
        a = jnp.pad(a, ((0, 0), (0, NCHM - NCHT0), (0, 0)),
                    constant_values=N)
        b = jnp.concatenate([v[E0:], jnp.full((E1 - (E - E0),), N,
                                              jnp.int32)])
        b = b.reshape(NTILES, NCHT1, CHUNK)
        b = jnp.pad(b, ((0, 0), (0, NCHM - NCHT1), (0, 0)),
                    constant_values=N)
        return jnp.concatenate([a, b], axis=0)

    srcp = split_edges(edge_index[0])
    dstp = split_edges(edge_index[1])
    xp = jnp.pad(x[:, 0], (0, NPAD - N)).reshape(1, NPAD)
    batchp = jnp.pad(batch, (0, NPAD - N),
                     constant_values=NG).reshape(1, NPAD)
    zeros1 = jnp.zeros((NPAD,), jnp.float32)
    zeros16 = jnp.zeros((NPAD, FP), jnp.float32)
    prop = _make_sc_prop()

    deg2 = _make_sc_degree()(dstp, zeros1)
    dinv, gcur = _k0(deg2.reshape(2, 1, NPAD), xp)

    specs = [(1, 8, W1, b1, g1, be1), (8, 16, W2, b2, g2, be2),
             (16, 32, W3, b3, g3, be3)]
    for i, (fi, fo, W, b, gm, be) in enumerate(specs):
        p = prop(srcp, dstp, gcur.T, zeros16)
        gcur = _TC_LAYER[i](jnp.transpose(p, (0, 2, 1)), gcur, dinv, W.T,
                            b.reshape(fo, 1), gm.reshape(fo, 1),
                            be.reshape(fo, 1))

    g4bf = gcur.astype(jnp.bfloat16).T
    zeros32 = jnp.zeros((NPAD, 32), jnp.bfloat16)
    p4 = _make_sc_prop(32, jnp.bfloat16)(srcp, dstp, g4bf, zeros32)
    h4 = _tc_layer4(jnp.transpose(p4, (0, 2, 1)), gcur, dinv, W4.T,
                    b4.reshape(64, 1))

    return _pool(h4, batchp, y_extra,
                 L1w[:NG], L1w[NG:], L1b.reshape(1, -1),
                 L2w, L2b.reshape(1, -1),
                 L3w, L3b.reshape(1, -1),
                 L4w, L4b.reshape(1, -1))


# speedup vs baseline: 1.4561x; 1.0333x over previous
import functools

import jax
import jax.numpy as jnp
from jax import lax
from jax.experimental import pallas as pl
from jax.experimental.pallas import tpu as pltpu
from jax.experimental.pallas import tpu_sc as plsc

N = 50000
NPAD = 50048
NG = 64
E = 800000
CHUNK = 128
NW = 32
NBUF = 4
NCHT0 = 128
NCHT1 = 264
NCHM = max(NCHT0, NCHT1)
E0 = 16 * NCHT0 * CHUNK
E1 = 16 * NCHT1 * CHUNK
NTILES = 16
RPT = NPAD // NTILES
_SQRT_HALF = 0.7071067811865476


def _gelu(t):
    return 0.5 * t * (1.0 + lax.erf(t * _SQRT_HALF))


def _sc_mesh():
    return plsc.VectorSubcoreMesh(core_axis_name="c", subcore_axis_name="s",
                                  num_cores=2, num_subcores=NTILES)


@functools.cache
def _make_sc_degree():
    return functools.partial(
        pl.kernel,
        out_type=jax.ShapeDtypeStruct((2, NPAD), jnp.float32),
        mesh=_sc_mesh(),
        scratch_types=[
            pltpu.VMEM((NCHM, CHUNK), jnp.int32),
            pltpu.VMEM((CHUNK,), jnp.float32),
            pltpu.VMEM_SHARED((NPAD,), jnp.float32),
        ] + [pltpu.SemaphoreType.DMA] * NBUF,
        compiler_params=pltpu.CompilerParams(use_tc_tiling_on_sc=False),
    )(_sc_degree_body)


def _sc_degree_body(dst_hbm, zeros_hbm, out_hbm, dst_v, ones_v, acc, *ssems):
    c = lax.axis_index("c")
    s = lax.axis_index("s")
    wid = c * NTILES + s
    base = s * RPT
    nblk = jnp.where(c == 0, NCHT0 // NBUF, NCHT1 // NBUF)
    pltpu.sync_copy(zeros_hbm.at[pl.ds(base, RPT)], acc.at[pl.ds(base, RPT)])
    pltpu.sync_copy(dst_hbm.at[wid], dst_v)
    for i in range(CHUNK // 16):
        ones_v[pl.ds(i * 16, 16)] = jnp.ones((16,), jnp.float32)
    plsc.subcore_barrier()

    def body(i, carry):
        j0 = i * NBUF
        for b in range(NBUF):
            pltpu.async_copy(ones_v, acc.at[dst_v.at[j0 + b]], ssems[b],
                             add=True)
        for b in range(NBUF):
            pltpu.make_async_copy(ones_v, acc.at[dst_v.at[j0 + b]],
                                  ssems[b]).wait()
        return carry

    lax.fori_loop(0, nblk, body, 0)
    plsc.subcore_barrier()
    pltpu.sync_copy(acc.at[pl.ds(base, RPT)], out_hbm.at[c, pl.ds(base, RPT)])


FP = 16


@functools.cache
def _make_sc_prop(F=FP, dtype=jnp.float32):
    feat = (F,)

    @functools.partial(
        pl.kernel,
        out_type=jax.ShapeDtypeStruct((2, NPAD) + feat, dtype),
        mesh=_sc_mesh(),
        scratch_types=[
            pltpu.VMEM((NCHM, CHUNK), jnp.int32),
            pltpu.VMEM((NCHM, CHUNK), jnp.int32),
            pltpu.VMEM_SHARED((NPAD,) + feat, dtype),
        ] + [pltpu.VMEM((CHUNK,) + feat, dtype)] * NBUF
          + [pltpu.SemaphoreType.DMA] * (2 * NBUF),
        compiler_params=pltpu.CompilerParams(use_tc_tiling_on_sc=False),
    )
    def prop(src_hbm, dst_hbm, g_hbm, zeros_hbm, out_hbm,
             src_v, dst_v, acc, *rest):
        bufs = rest[:NBUF]
        gsems = rest[NBUF:2 * NBUF]
        ssems = rest[2 * NBUF:3 * NBUF]
        c = lax.axis_index("c")
        s = lax.axis_index("s")
        wid = c * NTILES + s
        base = s * RPT
        nblk = jnp.where(c == 0, NCHT0 // NBUF, NCHT1 // NBUF)
        pltpu.sync_copy(zeros_hbm.at[pl.ds(base, RPT)],
                        acc.at[pl.ds(base, RPT)])
        pltpu.sync_copy(src_hbm.at[wid], src_v)
        pltpu.sync_copy(dst_hbm.at[wid], dst_v)
        plsc.subcore_barrier()

        for b in range(NBUF):
            pltpu.async_copy(g_hbm.at[src_v.at[b]], bufs[b], gsems[b])

        def body(i, carry):
            j0 = i * NBUF
            for b in range(NBUF):
                pltpu.make_async_copy(g_hbm.at[src_v.at[j0 + b]],
                                      bufs[b], gsems[b]).wait()
                pltpu.async_copy(bufs[b], acc.at[dst_v.at[j0 + b]],
                                 ssems[b], add=True)
            for b in range(NBUF):
                pltpu.make_async_copy(bufs[b], acc.at[dst_v.at[j0 + b]],
                                      ssems[b]).wait()
                pltpu.async_copy(g_hbm.at[src_v.at[j0 + NBUF + b]],
                                 bufs[b], gsems[b])
            return carry

        lax.fori_loop(0, nblk - 1, body, 0)
        j0 = (nblk - 1) * NBUF
        for b in range(NBUF):
            pltpu.make_async_copy(g_hbm.at[src_v.at[j0 + b]],
                                  bufs[b], gsems[b]).wait()
            pltpu.async_copy(bufs[b], acc.at[dst_v.at[j0 + b]],
                             ssems[b], add=True)
        for b in range(NBUF):
            pltpu.make_async_copy(bufs[b], acc.at[dst_v.at[j0 + b]],
                                  ssems[b]).wait()
        plsc.subcore_barrier()
        pltpu.sync_copy(acc.at[pl.ds(base, RPT)],
                        out_hbm.at[c, pl.ds(base, RPT)])

    return prop


def _lane_mask():
    i = lax.broadcasted_iota(jnp.int32, (1, NPAD), 1)
    return (i < N).astype(jnp.float32)


def _k0_body(deg_ref, x_ref, dinv_ref, g1_ref):
    deg = deg_ref[0] + deg_ref[1] + 1.0
    dinv = lax.rsqrt(deg) * _lane_mask()
    dinv_ref[...] = dinv
    g1_ref[...] = jnp.concatenate(
        [dinv * x_ref[...], jnp.zeros((FP - 1, NPAD), jnp.float32)], axis=0)


_k0 = pl.pallas_call(
    _k0_body,
    out_shape=(jax.ShapeDtypeStruct((1, NPAD), jnp.float32),
               jax.ShapeDtypeStruct((FP, NPAD), jnp.float32)),
)


def _bn_scale(h, use_bn, gm_ref, be_ref, dinv):
    if not use_bn:
        return h
    msk = _lane_mask()
    mu = jnp.sum(h * msk, axis=1, keepdims=True) * (1.0 / N)
    d = (h - mu) * msk
    var = jnp.sum(d * d, axis=1, keepdims=True) * (1.0 / N)
    h = (h - mu) * lax.rsqrt(var + 1e-5) * gm_ref[...] + be_ref[...]
    return h * dinv


def _layer_body(fi, fo, use_bn,
                p_ref, g_ref, dinv_ref, wt_ref, b_ref, gm_ref, be_ref,
                out_ref):
    agg = (p_ref[0, :fi] + p_ref[1, :fi] + g_ref[:fi]) * dinv_ref[...]
    if fi == 1:
        c = wt_ref[...] * agg
    else:
        c = lax.dot_general(wt_ref[...], agg, (((1,), (0,)), ((), ())),
                            preferred_element_type=jnp.float32)
    h = _gelu(c + b_ref[...])
    h = _bn_scale(h, use_bn, gm_ref, be_ref, dinv_ref[...])
    if fo < FP:
        h = jnp.concatenate(
            [h, jnp.zeros((FP - fo, NPAD), jnp.float32)], axis=0)
    out_ref[...] = h


def _make_tc_layer(fi, fo, use_bn):
    return pl.pallas_call(
        functools.partial(_layer_body, fi, fo, use_bn),
        out_shape=jax.ShapeDtypeStruct((max(fo, FP), NPAD), jnp.float32),
    )


def _layer4_body(p_ref, g_ref, dinv_ref, wt_ref, b_ref, out_ref):
    pf = p_ref[...].astype(jnp.float32)
    agg = (pf[0] + pf[1] + g_ref[...]) * dinv_ref[...]
    c = lax.dot_general(wt_ref[...], agg, (((1,), (0,)), ((), ())),
                        preferred_element_type=jnp.float32)
    out_ref[...] = _gelu(c + b_ref[...])


_tc_layer4 = pl.pallas_call(
    _layer4_body,
    out_shape=jax.ShapeDtypeStruct((64, NPAD), jnp.float32),
)

_TC_LAYER = [_make_tc_layer(1, 8, True), _make_tc_layer(8, 16, True),
             _make_tc_layer(16, 32, True)]


def _pool_body(h4_ref, batch_ref, y_ref,
               l1a_ref, l1b_ref, l1bias_ref, l2w_ref, l2b_ref,
               l3w_ref, l3b_ref, l4w_ref, l4b_ref, out_ref):
    gi = lax.broadcasted_iota(jnp.int32, (NG, 1), 0)
    onehot = (batch_ref[...] == gi).astype(jnp.float32)
    pooled = lax.dot_general(onehot, h4_ref[...], (((1,), (1,)), ((), ())),
                             preferred_element_type=jnp.float32)
    cnt = jnp.sum(onehot, axis=1, keepdims=True)
    z = pooled / jnp.maximum(cnt, 1.0)
    mm = lambda a, w: lax.dot_general(a, w, (((1,), (0,)), ((), ())),
                                      preferred_element_type=jnp.float32)
    z = _gelu(mm(z, l1a_ref[...]) + mm(y_ref[...], l1b_ref[...])
              + l1bias_ref[...])
    z = _gelu(mm(z, l2w_ref[...]) + l2b_ref[...])
    z = _gelu(mm(z, l3w_ref[...]) + l3b_ref[...])
    out_ref[...] = jax.nn.sigmoid(mm(z, l4w_ref[...]) + l4b_ref[...])


_pool = pl.pallas_call(
    _pool_body,
    out_shape=jax.ShapeDtypeStruct((NG, 2), jnp.float32),
)


def kernel(x, edge_index, batch, y_extra, W1, b1, W2, b2, W3, b3, W4, b4,
           g1, be1, g2, be2, g3, be3, L1w, L1b, L2w, L2b, L3w, L3b, L4w, L4b):
    def split_edges(v):
        a = v[:E0].reshape(NTILES, NCHT0, CHUNK)
        a = jnp.pad(a, ((0, 0), (0, NCHM - NCHT0), (0, 0)),
                    constant_values=N)
        b = jnp.concatenate([v[E0:], jnp.full((E1 - (E - E0),), N,
                                              jnp.int32)])
        b = b.reshape(NTILES, NCHT1, CHUNK)
        b = jnp.pad(b, ((0, 0), (0, NCHM - NCHT1), (0, 0)),
                    constant_values=N)
        return jnp.concatenate([a, b], axis=0)

    srcp = split_edges(edge_index[0])
    dstp = split_edges(edge_index[1])
    xp = jnp.pad(x[:, 0], (0, NPAD - N)).reshape(1, NPAD)
    batchp = jnp.pad(batch, (0, NPAD - N),
                     constant_values=NG).reshape(1, NPAD)
    zeros1 = jnp.zeros((NPAD,), jnp.float32)
    zeros16 = jnp.zeros((NPAD, FP), jnp.float32)
    prop = _make_sc_prop()

    deg2 = _make_sc_degree()(dstp, zeros1)
    dinv, gcur = _k0(deg2.reshape(2, 1, NPAD), xp)

    specs = [(1, 8, W1, b1, g1, be1), (8, 16, W2, b2, g2, be2),
             (16, 32, W3, b3, g3, be3)]
    for i, (fi, fo, W, b, gm, be) in enumerate(specs):
        p = prop(srcp, dstp, gcur.T, zeros16)
        gcur = _TC_LAYER[i](jnp.transpose(p, (0, 2, 1)), gcur, dinv, W.T,
                            b.reshape(fo, 1), gm.reshape(fo, 1),
                            be.reshape(fo, 1))

    g4bf = gcur.astype(jnp.bfloat16).T
    zeros32 = jnp.zeros((NPAD, 32), jnp.bfloat16)
    p4 = _make_sc_prop(32, jnp.bfloat16)(srcp, dstp, g4bf, zeros32)
    h4 = _tc_layer4(jnp.transpose(p4, (0, 2, 1)), gcur, dinv, W4.T,
                    b4.reshape(64, 1))

    return _pool(h4, batchp, y_extra,
                 L1w[:NG], L1w[NG:], L1b.reshape(1, -1),
                 L2w, L2b.reshape(1, -1),
                 L3w, L3b.reshape(1, -1),
                 L4w, L4b.reshape(1, -1))


# speedup vs baseline: 1.4971x; 1.0281x over previous
import functools

import jax
import jax.numpy as jnp
from jax import lax
from jax.experimental import pallas as pl
from jax.experimental.pallas import tpu as pltpu
from jax.experimental.pallas import tpu_sc as plsc

N = 50000
NPAD = 50048
NG = 64
E = 800000
CHUNK = 128
NW = 32
NBUF = 4
NCHT0 = 144
NCHT1 = 248
NCHM = max(NCHT0, NCHT1)
E0 = 16 * NCHT0 * CHUNK
E1 = 16 * NCHT1 * CHUNK
NTILES = 16
RPT = NPAD // NTILES
_SQRT_HALF = 0.7071067811865476


def _gelu(t):
    return 0.5 * t * (1.0 + lax.erf(t * _SQRT_HALF))


def _sc_mesh():
    return plsc.VectorSubcoreMesh(core_axis_name="c", subcore_axis_name="s",
                                  num_cores=2, num_subcores=NTILES)


@functools.cache
def _make_sc_degree():
    return functools.partial(
        pl.kernel,
        out_type=jax.ShapeDtypeStruct((2, NPAD), jnp.float32),
        mesh=_sc_mesh(),
        scratch_types=[
            pltpu.VMEM((NCHM, CHUNK), jnp.int32),
            pltpu.VMEM((CHUNK,), jnp.float32),
            pltpu.VMEM_SHARED((NPAD,), jnp.float32),
        ] + [pltpu.SemaphoreType.DMA] * NBUF,
        compiler_params=pltpu.CompilerParams(use_tc_tiling_on_sc=False),
    )(_sc_degree_body)


def _sc_degree_body(dst_hbm, zeros_hbm, out_hbm, dst_v, ones_v, acc, *ssems):
    c = lax.axis_index("c")
    s = lax.axis_index("s")
    wid = c * NTILES + s
    base = s * RPT
    nblk = jnp.where(c == 0, NCHT0 // NBUF, NCHT1 // NBUF)
    pltpu.sync_copy(zeros_hbm.at[pl.ds(base, RPT)], acc.at[pl.ds(base, RPT)])
    pltpu.sync_copy(dst_hbm.at[wid], dst_v)
    for i in range(CHUNK // 16):
        ones_v[pl.ds(i * 16, 16)] = jnp.ones((16,), jnp.float32)
    plsc.subcore_barrier()

    def body(i, carry):
        j0 = i * NBUF
        for b in range(NBUF):
            pltpu.async_copy(ones_v, acc.at[dst_v.at[j0 + b]], ssems[b],
                             add=True)
        for b in range(NBUF):
            pltpu.make_async_copy(ones_v, acc.at[dst_v.at[j0 + b]],
                                  ssems[b]).wait()
        return carry

    lax.fori_loop(0, nblk, body, 0)
    plsc.subcore_barrier()
    pltpu.sync_copy(acc.at[pl.ds(base, RPT)], out_hbm.at[c, pl.ds(base, RPT)])


FP = 16


@functools.cache
def _make_sc_prop(F=FP, dtype=jnp.float32):
    feat = (F,)

    @functools.partial(
        pl.kernel,
        out_type=jax.ShapeDtypeStruct((2, NPAD) + feat, dtype),
        mesh=_sc_mesh(),
        scratch_types=[
            pltpu.VMEM((NCHM, CHUNK), jnp.int32),
            pltpu.VMEM((NCHM, CHUNK), jnp.int32),
            pltpu.VMEM_SHARED((NPAD,) + feat, dtype),
        ] + [pltpu.VMEM((CHUNK,) + feat, dtype)] * NBUF
          + [pltpu.SemaphoreType.DMA] * (2 * NBUF),
        compiler_params=pltpu.CompilerParams(use_tc_tiling_on_sc=False),
    )
    def prop(src_hbm, dst_hbm, g_hbm, zeros_hbm, out_hbm,
             src_v, dst_v, acc, *rest):
        bufs = rest[:NBUF]
        gsems = rest[NBUF:2 * NBUF]
        ssems = rest[2 * NBUF:3 * NBUF]
        c = lax.axis_index("c")
        s = lax.axis_index("s")
        wid = c * NTILES + s
        base = s * RPT
        nblk = jnp.where(c == 0, NCHT0 // NBUF, NCHT1 // NBUF)
        pltpu.sync_copy(zeros_hbm.at[pl.ds(base, RPT)],
                        acc.at[pl.ds(base, RPT)])
        pltpu.sync_copy(src_hbm.at[wid], src_v)
        pltpu.sync_copy(dst_hbm.at[wid], dst_v)
        plsc.subcore_barrier()

        for b in range(NBUF):
            pltpu.async_copy(g_hbm.at[src_v.at[b]], bufs[b], gsems[b])

        def body(i, carry):
            j0 = i * NBUF
            for b in range(NBUF):
                pltpu.make_async_copy(g_hbm.at[src_v.at[j0 + b]],
                                      bufs[b], gsems[b]).wait()
                pltpu.async_copy(bufs[b], acc.at[dst_v.at[j0 + b]],
                                 ssems[b], add=True)
            for b in range(NBUF):
                pltpu.make_async_copy(bufs[b], acc.at[dst_v.at[j0 + b]],
                                      ssems[b]).wait()
                pltpu.async_copy(g_hbm.at[src_v.at[j0 + NBUF + b]],
                                 bufs[b], gsems[b])
            return carry

        lax.fori_loop(0, nblk - 1, body, 0)
        j0 = (nblk - 1) * NBUF
        for b in range(NBUF):
            pltpu.make_async_copy(g_hbm.at[src_v.at[j0 + b]],
                                  bufs[b], gsems[b]).wait()
            pltpu.async_copy(bufs[b], acc.at[dst_v.at[j0 + b]],
                             ssems[b], add=True)
        for b in range(NBUF):
            pltpu.make_async_copy(bufs[b], acc.at[dst_v.at[j0 + b]],
                                  ssems[b]).wait()
        plsc.subcore_barrier()
        pltpu.sync_copy(acc.at[pl.ds(base, RPT)],
                        out_hbm.at[c, pl.ds(base, RPT)])

    return prop


def _lane_mask():
    i = lax.broadcasted_iota(jnp.int32, (1, NPAD), 1)
    return (i < N).astype(jnp.float32)


def _k0_body(deg_ref, x_ref, dinv_ref, g1_ref):
    deg = deg_ref[0] + deg_ref[1] + 1.0
    dinv = lax.rsqrt(deg) * _lane_mask()
    dinv_ref[...] = dinv
    g1_ref[...] = jnp.concatenate(
        [dinv * x_ref[...], jnp.zeros((FP - 1, NPAD), jnp.float32)], axis=0)


_k0 = pl.pallas_call(
    _k0_body,
    out_shape=(jax.ShapeDtypeStruct((1, NPAD), jnp.float32),
               jax.ShapeDtypeStruct((FP, NPAD), jnp.float32)),
)


def _bn_scale(h, use_bn, gm_ref, be_ref, dinv):
    if not use_bn:
        return h
    msk = _lane_mask()
    mu = jnp.sum(h * msk, axis=1, keepdims=True) * (1.0 / N)
    d = (h - mu) * msk
    var = jnp.sum(d * d, axis=1, keepdims=True) * (1.0 / N)
    h = (h - mu) * lax.rsqrt(var + 1e-5) * gm_ref[...] + be_ref[...]
    return h * dinv


def _layer_body(fi, fo, use_bn,
                p_ref, g_ref, dinv_ref, wt_ref, b_ref, gm_ref, be_ref,
                out_ref):
    agg = (p_ref[0, :fi] + p_ref[1, :fi] + g_ref[:fi]) * dinv_ref[...]
    if fi == 1:
        c = wt_ref[...] * agg
    else:
        c = lax.dot_general(wt_ref[...], agg, (((1,), (0,)), ((), ())),
                            preferred_element_type=jnp.float32)
    h = _gelu(c + b_ref[...])
    h = _bn_scale(h, use_bn, gm_ref, be_ref, dinv_ref[...])
    if fo < FP:
        h = jnp.concatenate(
            [h, jnp.zeros((FP - fo, NPAD), jnp.float32)], axis=0)
    out_ref[...] = h


def _make_tc_layer(fi, fo, use_bn):
    return pl.pallas_call(
        functools.partial(_layer_body, fi, fo, use_bn),
        out_shape=jax.ShapeDtypeStruct((max(fo, FP), NPAD), jnp.float32),
    )


def _layer4_body(p_ref, g_ref, dinv_ref, wt_ref, b_ref, out_ref):
    pf = p_ref[...].astype(jnp.float32)
    agg = (pf[0] + pf[1] + g_ref[...]) * dinv_ref[...]
    c = lax.dot_general(wt_ref[...], agg, (((1,), (0,)), ((), ())),
                        preferred_element_type=jnp.float32)
    out_ref[...] = _gelu(c + b_ref[...])


_tc_layer4 = pl.pallas_call(
    _layer4_body,
    out_shape=jax.ShapeDtypeStruct((64, NPAD), jnp.float32),
)

_TC_LAYER = [_make_tc_layer(1, 8, True), _make_tc_layer(8, 16, True),
             _make_tc_layer(16, 32, True)]


def _pool_body(h4_ref, batch_ref, y_ref,
               l1a_ref, l1b_ref, l1bias_ref, l2w_ref, l2b_ref,
               l3w_ref, l3b_ref, l4w_ref, l4b_ref, out_ref):
    gi = lax.broadcasted_iota(jnp.int32, (NG, 1), 0)
    onehot = (batch_ref[...] == gi).astype(jnp.float32)
    pooled = lax.dot_general(onehot, h4_ref[...], (((1,), (1,)), ((), ())),
                             preferred_element_type=jnp.float32)
    cnt = jnp.sum(onehot, axis=1, keepdims=True)
    z = pooled / jnp.maximum(cnt, 1.0)
    mm = lambda a, w: lax.dot_general(a, w, (((1,), (0,)), ((), ())),
                                      preferred_element_type=jnp.float32)
    z = _gelu(mm(z, l1a_ref[...]) + mm(y_ref[...], l1b_ref[...])
              + l1bias_ref[...])
    z = _gelu(mm(z, l2w_ref[...]) + l2b_ref[...])
    z = _gelu(mm(z, l3w_ref[...]) + l3b_ref[...])
    out_ref[...] = jax.nn.sigmoid(mm(z, l4w_ref[...]) + l4b_ref[...])


_pool = pl.pallas_call(
    _pool_body,
    out_shape=jax.ShapeDtypeStruct((NG, 2), jnp.float32),
)


def kernel(x, edge_index, batch, y_extra, W1, b1, W2, b2, W3, b3, W4, b4,
           g1, be1, g2, be2, g3, be3, L1w, L1b, L2w, L2b, L3w, L3b, L4w, L4b):
    def split_edges(v):
        a = v[:E0].reshape(NTILES, NCHT0, CHUNK)
        a = jnp.pad(a, ((0, 0), (0, NCHM - NCHT0), (0, 0)),
                    constant_values=N)
        b = jnp.concatenate([v[E0:], jnp.full((E1 - (E - E0),), N,
                                              jnp.int32)])
        b = b.reshape(NTILES, NCHT1, CHUNK)
        b = jnp.pad(b, ((0, 0), (0, NCHM - NCHT1), (0, 0)),
                    constant_values=N)
        return jnp.concatenate([a, b], axis=0)

    srcp = split_edges(edge_index[0])
    dstp = split_edges(edge_index[1])
    xp = jnp.pad(x[:, 0], (0, NPAD - N)).reshape(1, NPAD)
    batchp = jnp.pad(batch, (0, NPAD - N),
                     constant_values=NG).reshape(1, NPAD)
    zeros1 = jnp.zeros((NPAD,), jnp.float32)
    zeros16 = jnp.zeros((NPAD, FP), jnp.float32)
    prop = _make_sc_prop()

    deg2 = _make_sc_degree()(dstp, zeros1)
    dinv, gcur = _k0(deg2.reshape(2, 1, NPAD), xp)

    specs = [(1, 8, W1, b1, g1, be1), (8, 16, W2, b2, g2, be2),
             (16, 32, W3, b3, g3, be3)]
    for i, (fi, fo, W, b, gm, be) in enumerate(specs):
        p = prop(srcp, dstp, gcur.T, zeros16)
        gcur = _TC_LAYER[i](jnp.transpose(p, (0, 2, 1)), gcur, dinv, W.T,
                            b.reshape(fo, 1), gm.reshape(fo, 1),
                            be.reshape(fo, 1))

    g4bf = gcur.astype(jnp.bfloat16).T
    zeros32 = jnp.zeros((NPAD, 32), jnp.bfloat16)
    p4 = _make_sc_prop(32, jnp.bfloat16)(srcp, dstp, g4bf, zeros32)
    h4 = _tc_layer4(jnp.transpose(p4, (0, 2, 1)), gcur, dinv, W4.T,
                    b4.reshape(64, 1))

    return _pool(h4, batchp, y_extra,
                 L1w[:NG], L1w[NG:], L1b.reshape(1, -1),
                 L2w, L2b.reshape(1, -1),
                 L3w, L3b.reshape(1, -1),
                 L4w, L4b.reshape(1, -1))


# speedup vs baseline: 1.4981x; 1.0007x over previous
import functools

import jax
import jax.numpy as jnp
from jax import lax
from jax.experimental import pallas as pl
from jax.experimental.pallas import tpu as pltpu
from jax.experimental.pallas import tpu_sc as plsc

N = 50000
NPAD = 50048
NG = 64
E = 800000
CHUNK = 128
NW = 32
NBUF = 4
NCHT0 = 160
NCHT1 = 232
NCHM = max(NCHT0, NCHT1)
E0 = 16 * NCHT0 * CHUNK
E1 = 16 * NCHT1 * CHUNK
NTILES = 16
RPT = NPAD // NTILES
_SQRT_HALF = 0.7071067811865476


def _gelu(t):
    return 0.5 * t * (1.0 + lax.erf(t * _SQRT_HALF))


def _sc_mesh():
    return plsc.VectorSubcoreMesh(core_axis_name="c", subcore_axis_name="s",
                                  num_cores=2, num_subcores=NTILES)


@functools.cache
def _make_sc_degree():
    return functools.partial(
        pl.kernel,
        out_type=jax.ShapeDtypeStruct((2, NPAD), jnp.float32),
        mesh=_sc_mesh(),
        scratch_types=[
            pltpu.VMEM((NCHM, CHUNK), jnp.int32),
            pltpu.VMEM((CHUNK,), jnp.float32),
            pltpu.VMEM_SHARED((NPAD,), jnp.float32),
        ] + [pltpu.SemaphoreType.DMA] * NBUF,
        compiler_params=pltpu.CompilerParams(use_tc_tiling_on_sc=False),
    )(_sc_degree_body)


def _sc_degree_body(dst_hbm, zeros_hbm, out_hbm, dst_v, ones_v, acc, *ssems):
    c = lax.axis_index("c")
    s = lax.axis_index("s")
    wid = c * NTILES + s
    base = s * RPT
    nblk = jnp.where(c == 0, NCHT0 // NBUF, NCHT1 // NBUF)
    pltpu.sync_copy(zeros_hbm.at[pl.ds(base, RPT)], acc.at[pl.ds(base, RPT)])
    pltpu.sync_copy(dst_hbm.at[wid], dst_v)
    for i in range(CHUNK // 16):
        ones_v[pl.ds(i * 16, 16)] = jnp.ones((16,), jnp.float32)
    plsc.subcore_barrier()

    def body(i, carry):
        j0 = i * NBUF
        for b in range(NBUF):
            pltpu.async_copy(ones_v, acc.at[dst_v.at[j0 + b]], ssems[b],
                             add=True)
        for b in range(NBUF):
            pltpu.make_async_copy(ones_v, acc.at[dst_v.at[j0 + b]],
                                  ssems[b]).wait()
        return carry

    lax.fori_loop(0, nblk, body, 0)
    plsc.subcore_barrier()
    pltpu.sync_copy(acc.at[pl.ds(base, RPT)], out_hbm.at[c, pl.ds(base, RPT)])


FP = 16


@functools.cache
def _make_sc_prop(F=FP, dtype=jnp.float32):
    feat = (F,)

    @functools.partial(
        pl.kernel,
        out_type=jax.ShapeDtypeStruct((2, NPAD) + feat, dtype),
        mesh=_sc_mesh(),
        scratch_types=[
            pltpu.VMEM((NCHM, CHUNK), jnp.int32),
            pltpu.VMEM((NCHM, CHUNK), jnp.int32),
            pltpu.VMEM_SHARED((NPAD,) + feat, dtype),
        ] + [pltpu.VMEM((CHUNK,) + feat, dtype)] * NBUF
          + [pltpu.SemaphoreType.DMA] * (2 * NBUF),
        compiler_params=pltpu.CompilerParams(use_tc_tiling_on_sc=False),
    )
    def prop(src_hbm, dst_hbm, g_hbm, zeros_hbm, out_hbm,
             src_v, dst_v, acc, *rest):
        bufs = rest[:NBUF]
        gsems = rest[NBUF:2 * NBUF]
        ssems = rest[2 * NBUF:3 * NBUF]
        c = lax.axis_index("c")
        s = lax.axis_index("s")
        wid = c * NTILES + s
        base = s * RPT
        nblk = jnp.where(c == 0, NCHT0 // NBUF, NCHT1 // NBUF)
        pltpu.sync_copy(zeros_hbm.at[pl.ds(base, RPT)],
                        acc.at[pl.ds(base, RPT)])
        pltpu.sync_copy(src_hbm.at[wid], src_v)
        pltpu.sync_copy(dst_hbm.at[wid], dst_v)
        plsc.subcore_barrier()

        for b in range(NBUF):
            pltpu.async_copy(g_hbm.at[src_v.at[b]], bufs[b], gsems[b])

        def body(i, carry):
            j0 = i * NBUF
            for b in range(NBUF):
                pltpu.make_async_copy(g_hbm.at[src_v.at[j0 + b]],
                                      bufs[b], gsems[b]).wait()
                pltpu.async_copy(bufs[b], acc.at[dst_v.at[j0 + b]],
                                 ssems[b], add=True)
            for b in range(NBUF):
                pltpu.make_async_copy(bufs[b], acc.at[dst_v.at[j0 + b]],
                                      ssems[b]).wait()
                pltpu.async_copy(g_hbm.at[src_v.at[j0 + NBUF + b]],
                                 bufs[b], gsems[b])
            return carry

        lax.fori_loop(0, nblk - 1, body, 0)
        j0 = (nblk - 1) * NBUF
        for b in range(NBUF):
            pltpu.make_async_copy(g_hbm.at[src_v.at[j0 + b]],
                                  bufs[b], gsems[b]).wait()
            pltpu.async_copy(bufs[b], acc.at[dst_v.at[j0 + b]],
                             ssems[b], add=True)
        for b in range(NBUF):
            pltpu.make_async_copy(bufs[b], acc.at[dst_v.at[j0 + b]],
                                  ssems[b]).wait()
        plsc.subcore_barrier()
        pltpu.sync_copy(acc.at[pl.ds(base, RPT)],
                        out_hbm.at[c, pl.ds(base, RPT)])

    return prop


def _lane_mask():
    i = lax.broadcasted_iota(jnp.int32, (1, NPAD), 1)
    return (i < N).astype(jnp.float32)


def _k0_body(deg_ref, x_ref, dinv_ref, g1_ref):
    deg = deg_ref[0] + deg_ref[1] + 1.0
    dinv = lax.rsqrt(deg) * _lane_mask()
    dinv_ref[...] = dinv
    g1_ref[...] = jnp.concatenate(
        [dinv * x_ref[...], jnp.zeros((FP - 1, NPAD), jnp.float32)], axis=0)


_k0 = pl.pallas_call(
    _k0_body,
    out_shape=(jax.ShapeDtypeStruct((1, NPAD), jnp.float32),
               jax.ShapeDtypeStruct((FP, NPAD), jnp.float32)),
)


def _bn_scale(h, use_bn, gm_ref, be_ref, dinv):
    if not use_bn:
        return h
    msk = _lane_mask()
    mu = jnp.sum(h * msk, axis=1, keepdims=True) * (1.0 / N)
    d = (h - mu) * msk
    var = jnp.sum(d * d, axis=1, keepdims=True) * (1.0 / N)
    h = (h - mu) * lax.rsqrt(var + 1e-5) * gm_ref[...] + be_ref[...]
    return h * dinv


def _layer_body(fi, fo, use_bn,
                p_ref, g_ref, dinv_ref, wt_ref, b_ref, gm_ref, be_ref,
                out_ref):
    agg = (p_ref[0, :fi] + p_ref[1, :fi] + g_ref[:fi]) * dinv_ref[...]
    if fi == 1:
        c = wt_ref[...] * agg
    else:
        c = lax.dot_general(wt_ref[...], agg, (((1,), (0,)), ((), ())),
                            preferred_element_type=jnp.float32)
    h = _gelu(c + b_ref[...])
    h = _bn_scale(h, use_bn, gm_ref, be_ref, dinv_ref[...])
    if fo < FP:
        h = jnp.concatenate(
            [h, jnp.zeros((FP - fo, NPAD), jnp.float32)], axis=0)
    out_ref[...] = h


def _make_tc_layer(fi, fo, use_bn):
    return pl.pallas_call(
        functools.partial(_layer_body, fi, fo, use_bn),
        out_shape=jax.ShapeDtypeStruct((max(fo, FP), NPAD), jnp.float32),
    )


def _layer4_body(p_ref, g_ref, dinv_ref, wt_ref, b_ref, out_ref):
    pf = p_ref[...].astype(jnp.float32)
    agg = (pf[0] + pf[1] + g_ref[...]) * dinv_ref[...]
    c = lax.dot_general(wt_ref[...], agg, (((1,), (0,)), ((), ())),
                        preferred_element_type=jnp.float32)
    out_ref[...] = _gelu(c + b_ref[...])


_tc_layer4 = pl.pallas_call(
    _layer4_body,
    out_shape=jax.ShapeDtypeStruct((64, NPAD), jnp.float32),
)

_TC_LAYER = [_make_tc_layer(1, 8, True), _make_tc_layer(8, 16, True),
             _make_tc_layer(16, 32, True)]


def _pool_body(h4_ref, batch_ref, y_ref,
               l1a_ref, l1b_ref, l1bias_ref, l2w_ref, l2b_ref,
               l3w_ref, l3b_ref, l4w_ref, l4b_ref, out_ref):
    gi = lax.broadcasted_iota(jnp.int32, (NG, 1), 0)
    onehot = (batch_ref[...] == gi).astype(jnp.float32)
    pooled = lax.dot_general(onehot, h4_ref[...], (((1,), (1,)), ((), ())),
                             preferred_element_type=jnp.float32)
    cnt = jnp.sum(onehot, axis=1, keepdims=True)
    z = pooled / jnp.maximum(cnt, 1.0)
    mm = lambda a, w: lax.dot_general(a, w, (((1,), (0,)), ((), ())),
                                      preferred_element_type=jnp.float32)
    z = _gelu(mm(z, l1a_ref[...]) + mm(y_ref[...], l1b_ref[...])
              + l1bias_ref[...])
    z = _gelu(mm(z, l2w_ref[...]) + l2b_ref[...])
    z = _gelu(mm(z, l3w_ref[...]) + l3b_ref[...])
    out_ref[...] = jax.nn.sigmoid(mm(z, l4w_ref[...]) + l4b_ref[...])


_pool = pl.pallas_call(
    _pool_body,
    out_shape=jax.ShapeDtypeStruct((NG, 2), jnp.float32),
)


def kernel(x, edge_index, batch, y_extra, W1, b1, W2, b2, W3, b3, W4, b4,
           g1, be1, g2, be2, g3, be3, L1w, L1b, L2w, L2b, L3w, L3b, L4w, L4b):
    def split_edges(v):
        a = v[:E0].reshape(NTILES, NCHT0, CHUNK)
        a = jnp.pad(a, ((0, 0), (0, NCHM - NCHT0), (0, 0)),
                    constant_values=N)
        b = jnp.concatenate([v[E0:], jnp.full((E1 - (E - E0),), N,
                                              jnp.int32)])
        b = b.reshape(NTILES, NCHT1, CHUNK)
        b = jnp.pad(b, ((0, 0), (0, NCHM - NCHT1), (0, 0)),
                    constant_values=N)
        return jnp.concatenate([a, b], axis=0)

    srcp = split_edges(edge_index[0])
    dstp = split_edges(edge_index[1])
    xp = jnp.pad(x[:, 0], (0, NPAD - N)).reshape(1, NPAD)
    batchp = jnp.pad(batch, (0, NPAD - N),
                     constant_values=NG).reshape(1, NPAD)
    zeros1 = jnp.zeros((NPAD,), jnp.float32)
    zeros16 = jnp.zeros((NPAD, FP), jnp.float32)
    prop = _make_sc_prop()

    deg2 = _make_sc_degree()(dstp, zeros1)
    dinv, gcur = _k0(deg2.reshape(2, 1, NPAD), xp)

    specs = [(1, 8, W1, b1, g1, be1), (8, 16, W2, b2, g2, be2),
             (16, 32, W3, b3, g3, be3)]
    for i, (fi, fo, W, b, gm, be) in enumerate(specs):
        p = prop(srcp, dstp, gcur.T, zeros16)
        gcur = _TC_LAYER[i](jnp.transpose(p, (0, 2, 1)), gcur, dinv, W.T,
                            b.reshape(fo, 1), gm.reshape(fo, 1),
                            be.reshape(fo, 1))

    g4bf = gcur.astype(jnp.bfloat16).T
    zeros32 = jnp.zeros((NPAD, 32), jnp.bfloat16)
    p4 = _make_sc_prop(32, jnp.bfloat16)(srcp, dstp, g4bf, zeros32)
    h4 = _tc_layer4(jnp.transpose(p4, (0, 2, 1)), gcur, dinv, W4.T,
                    b4.reshape(64, 1))

    return _pool(h4, batchp, y_extra,
                 L1w[:NG], L1w[NG:], L1b.reshape(1, -1),
                 L2w, L2b.reshape(1, -1),
                 L3w, L3b.reshape(1, -1),
                 L4w, L4b.reshape(1, -1))
